# baseline (speedup 1.0000x reference)
import functools

import jax
import jax.numpy as jnp
from jax import lax
from jax.experimental import pallas as pl
from jax.experimental.pallas import tpu as pltpu
from jax.experimental.pallas import tpu_sc as plsc

NC = 2
NS = 16
NW = NC * NS
CHUNK = 112
NCHUNK = 14
B_PER_W = CHUNK * NCHUNK
B_PAD = NW * B_PER_W
D = 64
EPS = 1e-5


def _sc_gather_stats(n_total, x_hbm, table_hbm, h_hbm, stats_hbm,
                     idx_v, rows_v, acc_v, gsem, wsem):
    wid = lax.axis_index("s") * NC + lax.axis_index("c")
    base = wid * B_PER_W

    pltpu.sync_copy(x_hbm.at[wid], idx_v)

    descs = []
    for j in range(NCHUNK):
        descs.append(pltpu.async_copy(
            table_hbm.at[idx_v.at[j]],
            rows_v.at[pl.ds(j * CHUNK, CHUNK)],
            gsem))
    for dsc in descs:
        dsc.wait()

    wdesc = pltpu.async_copy(rows_v, h_hbm.at[pl.ds(base, B_PER_W)], wsem)

    n_real = jnp.minimum(B_PER_W, n_total - base)
    n_grp = n_real // 8
    zero = jnp.zeros((16,), jnp.float32)

    def grp_body(g, carry):
        accs = list(carry)
        r0 = g * 8
        for rr in range(8):
            for c in range(4):
                v = rows_v[r0 + rr, pl.ds(16 * c, 16)]
                accs[c] = accs[c] + v
                accs[4 + c] = accs[4 + c] + v * v
        return tuple(accs)

    carry = lax.fori_loop(0, n_grp, grp_body, (zero,) * 8)

    def rem_body(r, carry):
        accs = list(carry)
        for c in range(4):
            v = rows_v[r, pl.ds(16 * c, 16)]
            accs[c] = accs[c] + v
            accs[4 + c] = accs[4 + c] + v * v
        return tuple(accs)

    carry = lax.fori_loop(n_grp * 8, n_real, rem_body, carry)

    for c in range(4):
        acc_v[0, pl.ds(16 * c, 16)] = carry[c]
        acc_v[1, pl.ds(16 * c, 16)] = carry[4 + c]
    pltpu.sync_copy(acc_v.at[0], stats_hbm.at[0, wid])
    pltpu.sync_copy(acc_v.at[1], stats_hbm.at[1, wid])

    wdesc.wait()


def _tc_affine(n_total, stats_ref, w_ref, b_ref, ms_ref, h_ref, o_ref):
    inv_n = 1.0 / n_total
    m1 = jnp.sum(stats_ref[0], axis=0) * inv_n
    q = jnp.sum(stats_ref[1], axis=0) * inv_n
    w = w_ref[0]
    b = b_ref[0]
    ms = ms_ref[0]
    v1 = q - ms * m1 * m1 * (2.0 - ms)
    r1 = lax.rsqrt(v1 + EPS)
    a1 = w * r1
    c1 = b - a1 * m1 * ms
    m2 = a1 * m1 + c1
    c2 = c1 - m2 * ms
    v2 = a1 * a1 * q + 2.0 * a1 * c2 * m1 + c2 * c2
    r2 = lax.rsqrt(v2 + EPS)
    a_f = w * r2 * a1
    c_f = w * r2 * c2 + b
    o_ref[...] = h_ref[...] * a_f[None, :] + c_f[None, :]


def kernel(x, edge_index, edge_weight, subG_nodes, batch_nodes,
           batch_nodes_mask, emb_table, gn_weight, gn_bias, gn_mean_scale):
    n_total = x.shape[0]
    xi = x.astype(jnp.int32)
    x_pad = jnp.pad(xi, (0, B_PAD - n_total)).reshape(NW, NCHUNK, CHUNK)

    mesh = plsc.VectorSubcoreMesh(core_axis_name="c", subcore_axis_name="s")
    sc_fn = pl.kernel(
        functools.partial(_sc_gather_stats, n_total),
        out_type=[
            jax.ShapeDtypeStruct((B_PAD, D), jnp.float32),
            jax.ShapeDtypeStruct((2, NW, D), jnp.float32),
        ],
        mesh=mesh,
        scratch_types=[
            pltpu.VMEM((NCHUNK, CHUNK), jnp.int32),
            pltpu.VMEM((B_PER_W, D), jnp.float32),
            pltpu.VMEM((2, D), jnp.float32),
            pltpu.SemaphoreType.DMA,
            pltpu.SemaphoreType.DMA,
        ],
        compiler_params=pltpu.CompilerParams(use_tc_tiling_on_sc=False),
    )
    h_pad, stats = sc_fn(x_pad, emb_table)

    grid = 8
    rb = B_PAD // grid
    out_pad = pl.pallas_call(
        functools.partial(_tc_affine, n_total),
        grid=(grid,),
        in_specs=[
            pl.BlockSpec((2, NW, D), lambda i: (0, 0, 0)),
            pl.BlockSpec((1, D), lambda i: (0, 0)),
            pl.BlockSpec((1, D), lambda i: (0, 0)),
            pl.BlockSpec((1, D), lambda i: (0, 0)),
            pl.BlockSpec((rb, D), lambda i: (i, 0)),
        ],
        out_specs=pl.BlockSpec((rb, D), lambda i: (i, 0)),
        out_shape=jax.ShapeDtypeStruct((B_PAD, D), jnp.float32),
    )(stats, gn_weight.reshape(1, D), gn_bias.reshape(1, D),
      gn_mean_scale.reshape(1, D), h_pad)

    return out_pad[:n_total]


# speedup vs baseline: 1.6349x; 1.6349x over previous
import functools

import jax
import jax.numpy as jnp
from jax import lax
from jax.experimental import pallas as pl
from jax.experimental.pallas import tpu as pltpu
from jax.experimental.pallas import tpu_sc as plsc

NC = 2
NS = 16
NW = NC * NS
B_PER_W = 1568
HB = B_PER_W // 2
B_PAD = NW * B_PER_W
D = 64
EPS = 1e-5


def _sc_gather_stats(n_total, x_hbm, table_hbm, h_hbm, stats_hbm,
                     idx_v, rows_v, acc_v, gsem, wsem):
    wid = lax.axis_index("s") * NC + lax.axis_index("c")
    base = wid * B_PER_W

    pltpu.sync_copy(x_hbm.at[wid], idx_v)

    lanes = lax.iota(jnp.int32, 16)

    def fire_grp(g, carry):
        vec = idx_v[pl.ds(g * 16, 16)].astype(jnp.float32)
        q0 = g * 8
        for j in range(16):
            s_idx = jnp.sum(jnp.where(lanes == j, vec, 0.0)).astype(jnp.int32)
            pltpu.async_copy(table_hbm.at[s_idx],
                             rows_v.at[q0 + j // 2, pl.ds(D * (j % 2), D)],
                             gsem)
        return carry

    lax.fori_loop(0, B_PER_W // 16, fire_grp, 0)
    pltpu.make_async_copy(h_hbm.at[pl.ds(wid * HB, HB)], rows_v, gsem).wait()

    wdesc = pltpu.async_copy(rows_v, h_hbm.at[pl.ds(wid * HB, HB)], wsem)

    n_real = jnp.minimum(B_PER_W, n_total - base)
    n_pair = n_real // 2
    n_grp = n_pair // 4
    zero = jnp.zeros((16,), jnp.float32)

    def grp_body(g, carry):
        accs = list(carry)
        q0 = g * 4
        for qq in range(4):
            for half in range(2):
                for c in range(4):
                    v = rows_v[q0 + qq, pl.ds(D * half + 16 * c, 16)]
                    accs[c] = accs[c] + v
                    accs[4 + c] = accs[4 + c] + v * v
        return tuple(accs)

    carry = lax.fori_loop(0, n_grp, grp_body, (zero,) * 8)

    def rem_body(q, carry):
        accs = list(carry)
        for half in range(2):
            for c in range(4):
                v = rows_v[q, pl.ds(D * half + 16 * c, 16)]
                accs[c] = accs[c] + v
                accs[4 + c] = accs[4 + c] + v * v
        return tuple(accs)

    carry = lax.fori_loop(n_grp * 4, n_pair, rem_body, carry)

    for c in range(4):
        acc_v[0, pl.ds(16 * c, 16)] = carry[c]
        acc_v[1, pl.ds(16 * c, 16)] = carry[4 + c]
    pltpu.sync_copy(acc_v.at[0], stats_hbm.at[0, wid])
    pltpu.sync_copy(acc_v.at[1], stats_hbm.at[1, wid])

    wdesc.wait()


def _tc_affine(n_total, stats_ref, w_ref, b_ref, ms_ref, h_ref, o_ref):
    inv_n = 1.0 / n_total
    m1 = jnp.sum(stats_ref[0], axis=0) * inv_n
    q = jnp.sum(stats_ref[1], axis=0) * inv_n
    w = w_ref[0]
    b = b_ref[0]
    ms = ms_ref[0]
    v1 = q - ms * m1 * m1 * (2.0 - ms)
    r1 = lax.rsqrt(v1 + EPS)
    a1 = w * r1
    c1 = b - a1 * m1 * ms
    m2 = a1 * m1 + c1
    c2 = c1 - m2 * ms
    v2 = a1 * a1 * q + 2.0 * a1 * c2 * m1 + c2 * c2
    r2 = lax.rsqrt(v2 + EPS)
    a_f = w * r2 * a1
    c_f = w * r2 * c2 + b
    a2 = jnp.concatenate([a_f, a_f])
    c2w = jnp.concatenate([c_f, c_f])
    o_ref[...] = h_ref[...] * a2[None, :] + c2w[None, :]


def kernel(x, edge_index, edge_weight, subG_nodes, batch_nodes,
           batch_nodes_mask, emb_table, gn_weight, gn_bias, gn_mean_scale):
    n_total = x.shape[0]
    xi = x.astype(jnp.int32)
    x_pad = jnp.pad(xi, (0, B_PAD - n_total)).reshape(NW, B_PER_W)

    mesh = plsc.VectorSubcoreMesh(core_axis_name="c", subcore_axis_name="s")
    sc_fn = pl.kernel(
        functools.partial(_sc_gather_stats, n_total),
        out_type=[
            jax.ShapeDtypeStruct((B_PAD // 2, 2 * D), jnp.float32),
            jax.ShapeDtypeStruct((2, NW, D), jnp.float32),
        ],
        mesh=mesh,
        scratch_types=[
            pltpu.VMEM((B_PER_W,), jnp.int32),
            pltpu.VMEM((HB, 2 * D), jnp.float32),
            pltpu.VMEM((2, D), jnp.float32),
            pltpu.SemaphoreType.DMA,
            pltpu.SemaphoreType.DMA,
        ],
        compiler_params=pltpu.CompilerParams(needs_layout_passes=False),
    )
    h_pack, stats = sc_fn(x_pad, emb_table)

    grid = 8
    rb2 = B_PAD // 2 // grid
    out_pad = pl.pallas_call(
        functools.partial(_tc_affine, n_total),
        grid=(grid,),
        in_specs=[
            pl.BlockSpec((2, NW, D), lambda i: (0, 0, 0)),
            pl.BlockSpec((1, D), lambda i: (0, 0)),
            pl.BlockSpec((1, D), lambda i: (0, 0)),
            pl.BlockSpec((1, D), lambda i: (0, 0)),
            pl.BlockSpec((rb2, 2 * D), lambda i: (i, 0)),
        ],
        out_specs=pl.BlockSpec((rb2, 2 * D), lambda i: (i, 0)),
        out_shape=jax.ShapeDtypeStruct((B_PAD // 2, 2 * D), jnp.float32),
    )(stats, gn_weight.reshape(1, D), gn_bias.reshape(1, D),
      gn_mean_scale.reshape(1, D), h_pack)

    return out_pad.reshape(B_PAD, D)[:n_total]
